# Initial kernel scaffold; baseline (speedup 1.0000x reference)
import jax
import jax.numpy as jnp
from jax.experimental import pallas as pl


def kernel(x, edge_index, batch, edge_attr, W_rel1, b_rel1, W_root1, W_rel2, b_rel2, W_root2, W_rel3, b_rel3, W_root3, w_c1, b_c1, w_c2, b_c2, w_c3, b_c3, w_c4, b_c4, W_m1, b_m1, W_m2, b_m2, W_m3, b_m3):
    raise NotImplementedError("write your pallas kernel here")


# speedup vs baseline: 5.9206x; 5.9206x over previous
import functools

import jax
import jax.numpy as jnp
from jax import lax
from jax.experimental import pallas as pl
from jax.experimental.pallas import tpu as pltpu
from jax.experimental.pallas import tpu_sc as plsc

N = 32768
E = 524288
BN = 1024

NC = 2
NS = 16
EB = 128
SBB = 32
EPS = E // NS
NSB = EPS // (EB * SBB)
ROWS_PER_TILE = N // NS


def _full_spec(shape):
    nd = len(shape)
    return pl.BlockSpec(shape, lambda *_: (0,) * nd)


def _sc_segsum_body(nq, tbl, src2d, dst2d, ea2d, out,
                    sidx, didx, eav, grow, srow, zrow, slab,
                    gs0, gs1, ss0, ss1, sts, std, ste):
    qps = nq // NC
    cid = lax.axis_index("c")
    sid = lax.axis_index("s")
    erow0 = sid * (EPS // EB)

    zf = jnp.zeros((16,), jnp.float32)

    def _zrow_init(j, _):
        zrow[j, pl.ds(0, 16)] = zf
        zrow[j, pl.ds(16, 16)] = zf
        return 0
    lax.fori_loop(0, EB, _zrow_init, 0)

    gsem = (gs0, gs1)
    ssem = (ss0, ss1)

    def stage(sb, half, sync):
        base = erow0 + sb * SBB
        if sync:
            pltpu.sync_copy(src2d.at[pl.ds(base, SBB)], sidx.at[pl.ds(half * SBB, SBB)])
            pltpu.sync_copy(dst2d.at[pl.ds(base, SBB)], didx.at[pl.ds(half * SBB, SBB)])
            pltpu.sync_copy(ea2d.at[pl.ds(base, SBB)], eav.at[pl.ds(half * SBB, SBB)])
        else:
            pltpu.async_copy(src2d.at[pl.ds(base, SBB)], sidx.at[pl.ds(half * SBB, SBB)], sts)
            pltpu.async_copy(dst2d.at[pl.ds(base, SBB)], didx.at[pl.ds(half * SBB, SBB)], std)
            pltpu.async_copy(ea2d.at[pl.ds(base, SBB)], eav.at[pl.ds(half * SBB, SBB)], ste)

    def wait_stage(half):
        pltpu.make_async_copy(src2d.at[pl.ds(0, SBB)], sidx.at[pl.ds(half * SBB, SBB)], sts).wait()
        pltpu.make_async_copy(dst2d.at[pl.ds(0, SBB)], didx.at[pl.ds(half * SBB, SBB)], std).wait()
        pltpu.make_async_copy(ea2d.at[pl.ds(0, SBB)], eav.at[pl.ds(half * SBB, SBB)], ste).wait()

    def fire_gather(half, blk, ph):
        pltpu.async_copy(tbl.at[sidx.at[half * SBB + blk]], grow.at[ph], gsem[ph])

    def wait_gather(ph):
        pltpu.make_async_copy(tbl.at[sidx.at[0]], grow.at[ph], gsem[ph]).wait()

    def fire_scatter(half, blk, ph):
        pltpu.async_copy(srow.at[ph], slab.at[didx.at[half * SBB + blk]], ssem[ph], add=True)

    def wait_scatter(ph):
        pltpu.make_async_copy(srow.at[ph], slab.at[didx.at[0]], ssem[ph]).wait()

    for qp in range(qps):
        q = cid * qps + qp
        qbase = q * N

        def _zero(k, _):
            pltpu.sync_copy(zrow, slab.at[pl.ds(sid * ROWS_PER_TILE + k * EB, EB)])
            return 0
        lax.fori_loop(0, ROWS_PER_TILE // EB, _zero, 0)
        plsc.subcore_barrier()

        stage(0, 0, sync=True)

        def sb_body(sb, _):
            half = lax.rem(sb, 2)

            @pl.when(sb > 0)
            def _():
                wait_stage(half)

            @pl.when(sb < NSB - 1)
            def _():
                stage(sb + 1, 1 - half, sync=False)

            qb16 = jnp.full((16,), qbase, jnp.int32)

            def _shift(r, _):
                row = half * SBB + r
                for c in range(EB // 16):
                    sidx[row, pl.ds(c * 16, 16)] = sidx[row, pl.ds(c * 16, 16)] + qb16
                return 0
            lax.fori_loop(0, SBB, _shift, 0)

            fire_gather(half, 0, 0)
            fire_gather(half, 1, 1)

            def pair_body(p, _):
                for ph in range(2):
                    blk = p * 2 + ph
                    wait_gather(ph)

                    @pl.when(blk >= 2)
                    def _():
                        wait_scatter(ph)

                    def _scale(jb, _):
                        ea16 = eav[half * SBB + blk, pl.ds(jb * 16, 16)]
                        for k in range(16):
                            j = jb * 16 + k
                            ev = jnp.full((16,), ea16[k])
                            srow[ph, j, pl.ds(0, 16)] = grow[ph, j, pl.ds(0, 16)] * ev
                            srow[ph, j, pl.ds(16, 16)] = grow[ph, j, pl.ds(16, 16)] * ev
                        return 0
                    lax.fori_loop(0, EB // 16, _scale, 0)

                    fire_scatter(half, blk, ph)

                    @pl.when(blk + 2 < SBB)
                    def _():
                        fire_gather(half, blk + 2, ph)
                return 0
            lax.fori_loop(0, SBB // 2, pair_body, 0)
            wait_scatter(0)
            wait_scatter(1)
            return 0
        lax.fori_loop(0, NSB, sb_body, 0)

        plsc.subcore_barrier()
        pltpu.sync_copy(slab.at[pl.ds(sid * ROWS_PER_TILE, ROWS_PER_TILE)],
                        out.at[pl.ds(qbase + sid * ROWS_PER_TILE, ROWS_PER_TILE)])
        plsc.subcore_barrier()


def _sc_segsum(tbl_q, src2d, dst2d, ea2d, nq):
    mesh = plsc.VectorSubcoreMesh(core_axis_name="c", subcore_axis_name="s",
                                  num_cores=NC, num_subcores=NS)
    f = pl.kernel(
        functools.partial(_sc_segsum_body, nq),
        out_type=jax.ShapeDtypeStruct((nq * N, 32), jnp.float32),
        mesh=mesh,
        compiler_params=pltpu.CompilerParams(use_tc_tiling_on_sc=False),
        scratch_types=[
            pltpu.VMEM((2 * SBB, EB), jnp.int32),
            pltpu.VMEM((2 * SBB, EB), jnp.int32),
            pltpu.VMEM((2 * SBB, EB), jnp.float32),
            pltpu.VMEM((2, EB, 32), jnp.float32),
            pltpu.VMEM((2, EB, 32), jnp.float32),
            pltpu.VMEM((EB, 32), jnp.float32),
            pltpu.VMEM_SHARED((N, 32), jnp.float32),
            pltpu.SemaphoreType.DMA,
            pltpu.SemaphoreType.DMA,
            pltpu.SemaphoreType.DMA,
            pltpu.SemaphoreType.DMA,
            pltpu.SemaphoreType.DMA,
            pltpu.SemaphoreType.DMA,
            pltpu.SemaphoreType.DMA,
        ],
    )
    return f(tbl_q, src2d, dst2d, ea2d)


def _t1_body(agg_ref, x_ref, wr1_ref, br1_ref, wo1_ref, wo2_ref,
             h1q_ref, r2_ref):
    agg = jnp.concatenate([agg_ref[q] for q in range(4)], axis=1)
    h1 = jnp.tanh(
        jnp.dot(agg, wr1_ref[...], preferred_element_type=jnp.float32)
        + br1_ref[...]
        + jnp.dot(x_ref[...], wo1_ref[...], preferred_element_type=jnp.float32))
    for q in range(8):
        h1q_ref[q] = h1[:, q * 32:(q + 1) * 32]
    r2_ref[...] = jnp.dot(h1, wo2_ref[...], preferred_element_type=jnp.float32)


def _t1(agg1, x, W_rel1, b_rel1, W_root1, W_root2):
    return pl.pallas_call(
        _t1_body,
        grid=(N // BN,),
        in_specs=[
            pl.BlockSpec((4, BN, 32), lambda i: (0, i, 0)),
            pl.BlockSpec((BN, 128), lambda i: (i, 0)),
            _full_spec((128, 256)),
            _full_spec((1, 256)),
            _full_spec((128, 256)),
            _full_spec((256, 128)),
        ],
        out_specs=[
            pl.BlockSpec((8, BN, 32), lambda i: (0, i, 0)),
            pl.BlockSpec((BN, 128), lambda i: (i, 0)),
        ],
        out_shape=[
            jax.ShapeDtypeStruct((8, N, 32), jnp.float32),
            jax.ShapeDtypeStruct((N, 128), jnp.float32),
        ],
    )(agg1, x, W_rel1, b_rel1.reshape(1, 256), W_root1, W_root2)


def _t2_body(agg_ref, r2_ref, wr2_ref, br2_ref, wo3_ref, h2q_ref, r3_ref):
    agg = jnp.concatenate([agg_ref[q] for q in range(8)], axis=1)
    h2 = jnp.tanh(
        jnp.dot(agg, wr2_ref[...], preferred_element_type=jnp.float32)
        + br2_ref[...] + r2_ref[...])
    for q in range(4):
        h2q_ref[q] = h2[:, q * 32:(q + 1) * 32]
    r3_ref[...] = jnp.dot(h2, wo3_ref[...], preferred_element_type=jnp.float32)


def _t2(agg2, r2, W_rel2, b_rel2, W_root3):
    return pl.pallas_call(
        _t2_body,
        grid=(N // BN,),
        in_specs=[
            pl.BlockSpec((8, BN, 32), lambda i: (0, i, 0)),
            pl.BlockSpec((BN, 128), lambda i: (i, 0)),
            _full_spec((256, 128)),
            _full_spec((1, 128)),
            _full_spec((128, 64)),
        ],
        out_specs=[
            pl.BlockSpec((4, BN, 32), lambda i: (0, i, 0)),
            pl.BlockSpec((BN, 64), lambda i: (i, 0)),
        ],
        out_shape=[
            jax.ShapeDtypeStruct((4, N, 32), jnp.float32),
            jax.ShapeDtypeStruct((N, 64), jnp.float32),
        ],
    )(agg2, r2, W_rel2, b_rel2.reshape(1, 128), W_root3)


def _t3_body(agg_ref, r3_ref, wr3_ref, br3_ref, c1_ref, c2_ref, c3_ref,
             c4_ref, bc_ref, s_ref):
    agg = jnp.concatenate([agg_ref[q] for q in range(4)], axis=1)
    h3 = jnp.tanh(
        jnp.dot(agg, wr3_ref[...], preferred_element_type=jnp.float32)
        + br3_ref[...] + r3_ref[...])
    bc = bc_ref[...]
    t = jax.nn.relu(jnp.dot(h3, c1_ref[...], preferred_element_type=jnp.float32)
                    + bc[0, 0])
    t = jax.nn.relu(jnp.dot(t, c2_ref[...], preferred_element_type=jnp.float32)
                    + bc[0, 1])
    t = jax.nn.relu(jnp.dot(t, c3_ref[...], preferred_element_type=jnp.float32)
                    + bc[0, 2])
    t = jax.nn.relu(jnp.dot(t, c4_ref[...], preferred_element_type=jnp.float32)
                    + bc[0, 3])
    s_ref[...] = t


def _t3(agg3, r3, W_rel3, b_rel3, C1, C2, C3, C4, bc):
    return pl.pallas_call(
        _t3_body,
        grid=(N // BN,),
        in_specs=[
            pl.BlockSpec((4, BN, 32), lambda i: (0, i, 0)),
            pl.BlockSpec((BN, 64), lambda i: (i, 0)),
            _full_spec((128, 64)),
            _full_spec((1, 64)),
            _full_spec((64, 31)),
            _full_spec((31, 15)),
            _full_spec((15, 6)),
            _full_spec((6, 1)),
            _full_spec((1, 4)),
        ],
        out_specs=pl.BlockSpec((BN, 1), lambda i: (i, 0)),
        out_shape=jax.ShapeDtypeStruct((N, 1), jnp.float32),
    )(agg3, r3, W_rel3, b_rel3.reshape(1, 64), C1, C2, C3, C4, bc)


def _t4_body(g_ref, w1_ref, b1_ref, w2_ref, b2_ref, w3_ref, b3_ref, o_ref):
    o = jax.nn.relu(jnp.dot(g_ref[...], w1_ref[...],
                            preferred_element_type=jnp.float32) + b1_ref[...])
    o = jax.nn.relu(jnp.dot(o, w2_ref[...],
                            preferred_element_type=jnp.float32) + b2_ref[...])
    o_ref[...] = (jnp.dot(o, w3_ref[...], preferred_element_type=jnp.float32)
                  + b3_ref[...])


def _t4(g, W_m1, b_m1, W_m2, b_m2, W_m3, b_m3):
    B = g.shape[0]
    return pl.pallas_call(
        _t4_body,
        in_specs=[_full_spec((B, 32)), _full_spec((32, 16)), _full_spec((1, 16)),
                  _full_spec((16, 8)), _full_spec((1, 8)),
                  _full_spec((8, 1)), _full_spec((1, 1))],
        out_specs=_full_spec((B, 1)),
        out_shape=jax.ShapeDtypeStruct((B, 1), jnp.float32),
    )(g, W_m1, b_m1.reshape(1, 16), W_m2, b_m2.reshape(1, 8),
      W_m3, b_m3.reshape(1, 1))


def _band(w, L_in, stride):
    k = w.shape[0]
    L_out = (L_in - k) // stride + 1
    i = jnp.arange(L_in)[:, None]
    j = jnp.arange(L_out)[None, :]
    tap = i - stride * j
    m = (tap >= 0) & (tap < k)
    return jnp.where(m, w[jnp.clip(tap, 0, k - 1)], 0.0).astype(jnp.float32)


def kernel(x, edge_index, batch, edge_attr,
           W_rel1, b_rel1, W_root1, W_rel2, b_rel2, W_root2,
           W_rel3, b_rel3, W_root3,
           w_c1, b_c1, w_c2, b_c2, w_c3, b_c3, w_c4, b_c4,
           W_m1, b_m1, W_m2, b_m2, W_m3, b_m3):
    src2d = edge_index[0].reshape(E // EB, EB)
    dst2d = edge_index[1].reshape(E // EB, EB)
    ea2d = edge_attr.reshape(E // EB, EB)

    x_q = jnp.transpose(x.reshape(N, 4, 32), (1, 0, 2)).reshape(4 * N, 32)
    agg1 = _sc_segsum(x_q, src2d, dst2d, ea2d, 4).reshape(4, N, 32)
    h1q, r2 = _t1(agg1, x, W_rel1, b_rel1, W_root1, W_root2)

    agg2 = _sc_segsum(h1q.reshape(8 * N, 32), src2d, dst2d, ea2d, 8).reshape(8, N, 32)
    h2q, r3 = _t2(agg2, r2, W_rel2, b_rel2, W_root3)

    agg3 = _sc_segsum(h2q.reshape(4 * N, 32), src2d, dst2d, ea2d, 4).reshape(4, N, 32)

    C1 = _band(w_c1, 64, 2)
    C2 = _band(w_c2, 31, 2)
    C3 = _band(w_c3, 15, 2)
    C4 = _band(w_c4, 6, 1)
    bc = jnp.stack([b_c1[0], b_c2[0], b_c3[0], b_c4[0]]).reshape(1, 4)
    s = _t3(agg3, r3, W_rel3, b_rel3, C1, C2, C3, C4, bc)

    g = s.reshape(N // 32, 32)
    return _t4(g, W_m1, b_m1, W_m2, b_m2, W_m3, b_m3)


# speedup vs baseline: 7.3262x; 1.2374x over previous
import functools

import jax
import jax.numpy as jnp
from jax import lax
from jax.experimental import pallas as pl
from jax.experimental.pallas import tpu as pltpu
from jax.experimental.pallas import tpu_sc as plsc

N = 32768
E = 524288
BN = 1024

NC = 2
NS = 16
EB = 128
SBB = 32
EPS = E // NS
NSB = EPS // (EB * SBB)
ROWS_PER_TILE = N // NS


def _full_spec(shape):
    nd = len(shape)
    return pl.BlockSpec(shape, lambda *_: (0,) * nd)


def _sc_segsum_body(nq, tbl, src2d, dst2d, ea2d, out,
                    sidx, didx, eav, grow, srow, zrow, slab,
                    gs0, gs1, gs2, gs3, ss0, ss1, ss2, ss3, sts, std, ste):
    qps = nq // NC
    cid = lax.axis_index("c")
    sid = lax.axis_index("s")
    erow0 = sid * (EPS // EB)

    zf = jnp.zeros((16,), jnp.float32)

    def _zrow_init(j, _):
        zrow[j, pl.ds(0, 16)] = zf
        zrow[j, pl.ds(16, 16)] = zf
        return 0
    lax.fori_loop(0, EB, _zrow_init, 0)

    gsem = (gs0, gs1, gs2, gs3)
    ssem = (ss0, ss1, ss2, ss3)

    def stage(sb, half, sync):
        base = erow0 + sb * SBB
        if sync:
            pltpu.sync_copy(src2d.at[pl.ds(base, SBB)], sidx.at[pl.ds(half * SBB, SBB)])
            pltpu.sync_copy(dst2d.at[pl.ds(base, SBB)], didx.at[pl.ds(half * SBB, SBB)])
            pltpu.sync_copy(ea2d.at[pl.ds(base, SBB)], eav.at[pl.ds(half * SBB, SBB)])
        else:
            pltpu.async_copy(src2d.at[pl.ds(base, SBB)], sidx.at[pl.ds(half * SBB, SBB)], sts)
            pltpu.async_copy(dst2d.at[pl.ds(base, SBB)], didx.at[pl.ds(half * SBB, SBB)], std)
            pltpu.async_copy(ea2d.at[pl.ds(base, SBB)], eav.at[pl.ds(half * SBB, SBB)], ste)

    def wait_stage(half):
        pltpu.make_async_copy(src2d.at[pl.ds(0, SBB)], sidx.at[pl.ds(half * SBB, SBB)], sts).wait()
        pltpu.make_async_copy(dst2d.at[pl.ds(0, SBB)], didx.at[pl.ds(half * SBB, SBB)], std).wait()
        pltpu.make_async_copy(ea2d.at[pl.ds(0, SBB)], eav.at[pl.ds(half * SBB, SBB)], ste).wait()

    def fire_gather(half, blk, ph):
        pltpu.async_copy(tbl.at[sidx.at[half * SBB + blk]], grow.at[ph], gsem[ph])

    def wait_gather(ph):
        pltpu.make_async_copy(tbl.at[sidx.at[0]], grow.at[ph], gsem[ph]).wait()

    def fire_scatter(half, blk, ph):
        pltpu.async_copy(srow.at[ph], slab.at[didx.at[half * SBB + blk]], ssem[ph], add=True)

    def wait_scatter(ph):
        pltpu.make_async_copy(srow.at[ph], slab.at[didx.at[0]], ssem[ph]).wait()

    for qp in range(qps):
        q = cid * qps + qp
        qbase = q * N

        def _zero(k, _):
            pltpu.sync_copy(zrow, slab.at[pl.ds(sid * ROWS_PER_TILE + k * EB, EB)])
            return 0
        lax.fori_loop(0, ROWS_PER_TILE // EB, _zero, 0)
        plsc.subcore_barrier()

        stage(0, 0, sync=True)

        def sb_body(sb, _):
            half = lax.rem(sb, 2)

            @pl.when(sb > 0)
            def _():
                wait_stage(half)

            @pl.when(sb < NSB - 1)
            def _():
                stage(sb + 1, 1 - half, sync=False)

            qb16 = jnp.full((16,), qbase, jnp.int32)

            def _shift(r, _):
                row = half * SBB + r
                for c in range(EB // 16):
                    sidx[row, pl.ds(c * 16, 16)] = sidx[row, pl.ds(c * 16, 16)] + qb16
                return 0
            lax.fori_loop(0, SBB, _shift, 0)

            for ph in range(4):
                fire_gather(half, ph, ph)

            def quad_body(p, _):
                for ph in range(4):
                    blk = p * 4 + ph
                    wait_gather(ph)

                    @pl.when(blk >= 4)
                    def _():
                        wait_scatter(ph)

                    def _scale(jb, _):
                        ea16 = eav[half * SBB + blk, pl.ds(jb * 16, 16)]
                        for k in range(16):
                            j = jb * 16 + k
                            ev = jnp.full((16,), ea16[k])
                            srow[ph, j, pl.ds(0, 16)] = grow[ph, j, pl.ds(0, 16)] * ev
                            srow[ph, j, pl.ds(16, 16)] = grow[ph, j, pl.ds(16, 16)] * ev
                        return 0
                    lax.fori_loop(0, EB // 16, _scale, 0)

                    fire_scatter(half, blk, ph)

                    @pl.when(blk + 4 < SBB)
                    def _():
                        fire_gather(half, blk + 4, ph)
                return 0
            lax.fori_loop(0, SBB // 4, quad_body, 0)
            for ph in range(4):
                wait_scatter(ph)
            return 0
        lax.fori_loop(0, NSB, sb_body, 0)

        plsc.subcore_barrier()
        pltpu.sync_copy(slab.at[pl.ds(sid * ROWS_PER_TILE, ROWS_PER_TILE)],
                        out.at[pl.ds(qbase + sid * ROWS_PER_TILE, ROWS_PER_TILE)])
        plsc.subcore_barrier()


def _sc_segsum(tbl_q, src2d, dst2d, ea2d, nq):
    mesh = plsc.VectorSubcoreMesh(core_axis_name="c", subcore_axis_name="s",
                                  num_cores=NC, num_subcores=NS)
    f = pl.kernel(
        functools.partial(_sc_segsum_body, nq),
        out_type=jax.ShapeDtypeStruct((nq * N, 32), jnp.float32),
        mesh=mesh,
        compiler_params=pltpu.CompilerParams(use_tc_tiling_on_sc=False),
        scratch_types=[
            pltpu.VMEM((2 * SBB, EB), jnp.int32),
            pltpu.VMEM((2 * SBB, EB), jnp.int32),
            pltpu.VMEM((2 * SBB, EB), jnp.float32),
            pltpu.VMEM((4, EB, 32), jnp.float32),
            pltpu.VMEM((4, EB, 32), jnp.float32),
            pltpu.VMEM((EB, 32), jnp.float32),
            pltpu.VMEM_SHARED((N, 32), jnp.float32),
            pltpu.SemaphoreType.DMA,
            pltpu.SemaphoreType.DMA,
            pltpu.SemaphoreType.DMA,
            pltpu.SemaphoreType.DMA,
            pltpu.SemaphoreType.DMA,
            pltpu.SemaphoreType.DMA,
            pltpu.SemaphoreType.DMA,
            pltpu.SemaphoreType.DMA,
            pltpu.SemaphoreType.DMA,
            pltpu.SemaphoreType.DMA,
            pltpu.SemaphoreType.DMA,
        ],
    )
    return f(tbl_q, src2d, dst2d, ea2d)


def _t1_body(agg_ref, x_ref, wr1_ref, br1_ref, wo1_ref, wo2_ref,
             h1q_ref, r2_ref):
    agg = jnp.concatenate([agg_ref[q] for q in range(4)], axis=1)
    h1 = jnp.tanh(
        jnp.dot(agg, wr1_ref[...], preferred_element_type=jnp.float32)
        + br1_ref[...]
        + jnp.dot(x_ref[...], wo1_ref[...], preferred_element_type=jnp.float32))
    for q in range(8):
        h1q_ref[q] = h1[:, q * 32:(q + 1) * 32]
    r2_ref[...] = jnp.dot(h1, wo2_ref[...], preferred_element_type=jnp.float32)


def _t1(agg1, x, W_rel1, b_rel1, W_root1, W_root2):
    return pl.pallas_call(
        _t1_body,
        grid=(N // BN,),
        in_specs=[
            pl.BlockSpec((4, BN, 32), lambda i: (0, i, 0)),
            pl.BlockSpec((BN, 128), lambda i: (i, 0)),
            _full_spec((128, 256)),
            _full_spec((1, 256)),
            _full_spec((128, 256)),
            _full_spec((256, 128)),
        ],
        out_specs=[
            pl.BlockSpec((8, BN, 32), lambda i: (0, i, 0)),
            pl.BlockSpec((BN, 128), lambda i: (i, 0)),
        ],
        out_shape=[
            jax.ShapeDtypeStruct((8, N, 32), jnp.float32),
            jax.ShapeDtypeStruct((N, 128), jnp.float32),
        ],
    )(agg1, x, W_rel1, b_rel1.reshape(1, 256), W_root1, W_root2)


def _t2_body(agg_ref, r2_ref, wr2_ref, br2_ref, wo3_ref, h2q_ref, r3_ref):
    agg = jnp.concatenate([agg_ref[q] for q in range(8)], axis=1)
    h2 = jnp.tanh(
        jnp.dot(agg, wr2_ref[...], preferred_element_type=jnp.float32)
        + br2_ref[...] + r2_ref[...])
    for q in range(4):
        h2q_ref[q] = h2[:, q * 32:(q + 1) * 32]
    r3_ref[...] = jnp.dot(h2, wo3_ref[...], preferred_element_type=jnp.float32)


def _t2(agg2, r2, W_rel2, b_rel2, W_root3):
    return pl.pallas_call(
        _t2_body,
        grid=(N // BN,),
        in_specs=[
            pl.BlockSpec((8, BN, 32), lambda i: (0, i, 0)),
            pl.BlockSpec((BN, 128), lambda i: (i, 0)),
            _full_spec((256, 128)),
            _full_spec((1, 128)),
            _full_spec((128, 64)),
        ],
        out_specs=[
            pl.BlockSpec((4, BN, 32), lambda i: (0, i, 0)),
            pl.BlockSpec((BN, 64), lambda i: (i, 0)),
        ],
        out_shape=[
            jax.ShapeDtypeStruct((4, N, 32), jnp.float32),
            jax.ShapeDtypeStruct((N, 64), jnp.float32),
        ],
    )(agg2, r2, W_rel2, b_rel2.reshape(1, 128), W_root3)


def _t3_body(agg_ref, r3_ref, wr3_ref, br3_ref, c1_ref, c2_ref, c3_ref,
             c4_ref, bc_ref, s_ref):
    agg = jnp.concatenate([agg_ref[q] for q in range(4)], axis=1)
    h3 = jnp.tanh(
        jnp.dot(agg, wr3_ref[...], preferred_element_type=jnp.float32)
        + br3_ref[...] + r3_ref[...])
    bc = bc_ref[...]
    t = jax.nn.relu(jnp.dot(h3, c1_ref[...], preferred_element_type=jnp.float32)
                    + bc[0, 0])
    t = jax.nn.relu(jnp.dot(t, c2_ref[...], preferred_element_type=jnp.float32)
                    + bc[0, 1])
    t = jax.nn.relu(jnp.dot(t, c3_ref[...], preferred_element_type=jnp.float32)
                    + bc[0, 2])
    t = jax.nn.relu(jnp.dot(t, c4_ref[...], preferred_element_type=jnp.float32)
                    + bc[0, 3])
    s_ref[...] = t


def _t3(agg3, r3, W_rel3, b_rel3, C1, C2, C3, C4, bc):
    return pl.pallas_call(
        _t3_body,
        grid=(N // BN,),
        in_specs=[
            pl.BlockSpec((4, BN, 32), lambda i: (0, i, 0)),
            pl.BlockSpec((BN, 64), lambda i: (i, 0)),
            _full_spec((128, 64)),
            _full_spec((1, 64)),
            _full_spec((64, 31)),
            _full_spec((31, 15)),
            _full_spec((15, 6)),
            _full_spec((6, 1)),
            _full_spec((1, 4)),
        ],
        out_specs=pl.BlockSpec((BN, 1), lambda i: (i, 0)),
        out_shape=jax.ShapeDtypeStruct((N, 1), jnp.float32),
    )(agg3, r3, W_rel3, b_rel3.reshape(1, 64), C1, C2, C3, C4, bc)


def _t4_body(g_ref, w1_ref, b1_ref, w2_ref, b2_ref, w3_ref, b3_ref, o_ref):
    o = jax.nn.relu(jnp.dot(g_ref[...], w1_ref[...],
                            preferred_element_type=jnp.float32) + b1_ref[...])
    o = jax.nn.relu(jnp.dot(o, w2_ref[...],
                            preferred_element_type=jnp.float32) + b2_ref[...])
    o_ref[...] = (jnp.dot(o, w3_ref[...], preferred_element_type=jnp.float32)
                  + b3_ref[...])


def _t4(g, W_m1, b_m1, W_m2, b_m2, W_m3, b_m3):
    B = g.shape[0]
    return pl.pallas_call(
        _t4_body,
        in_specs=[_full_spec((B, 32)), _full_spec((32, 16)), _full_spec((1, 16)),
                  _full_spec((16, 8)), _full_spec((1, 8)),
                  _full_spec((8, 1)), _full_spec((1, 1))],
        out_specs=_full_spec((B, 1)),
        out_shape=jax.ShapeDtypeStruct((B, 1), jnp.float32),
    )(g, W_m1, b_m1.reshape(1, 16), W_m2, b_m2.reshape(1, 8),
      W_m3, b_m3.reshape(1, 1))


def _band(w, L_in, stride):
    k = w.shape[0]
    L_out = (L_in - k) // stride + 1
    i = jnp.arange(L_in)[:, None]
    j = jnp.arange(L_out)[None, :]
    tap = i - stride * j
    m = (tap >= 0) & (tap < k)
    return jnp.where(m, w[jnp.clip(tap, 0, k - 1)], 0.0).astype(jnp.float32)


def kernel(x, edge_index, batch, edge_attr,
           W_rel1, b_rel1, W_root1, W_rel2, b_rel2, W_root2,
           W_rel3, b_rel3, W_root3,
           w_c1, b_c1, w_c2, b_c2, w_c3, b_c3, w_c4, b_c4,
           W_m1, b_m1, W_m2, b_m2, W_m3, b_m3):
    src2d = edge_index[0].reshape(E // EB, EB)
    dst2d = edge_index[1].reshape(E // EB, EB)
    ea2d = edge_attr.reshape(E // EB, EB)

    x_q = jnp.transpose(x.reshape(N, 4, 32), (1, 0, 2)).reshape(4 * N, 32)
    agg1 = _sc_segsum(x_q, src2d, dst2d, ea2d, 4).reshape(4, N, 32)
    h1q, r2 = _t1(agg1, x, W_rel1, b_rel1, W_root1, W_root2)

    agg2 = _sc_segsum(h1q.reshape(8 * N, 32), src2d, dst2d, ea2d, 8).reshape(8, N, 32)
    h2q, r3 = _t2(agg2, r2, W_rel2, b_rel2, W_root3)

    agg3 = _sc_segsum(h2q.reshape(4 * N, 32), src2d, dst2d, ea2d, 4).reshape(4, N, 32)

    C1 = _band(w_c1, 64, 2)
    C2 = _band(w_c2, 31, 2)
    C3 = _band(w_c3, 15, 2)
    C4 = _band(w_c4, 6, 1)
    bc = jnp.stack([b_c1[0], b_c2[0], b_c3[0], b_c4[0]]).reshape(1, 4)
    s = _t3(agg3, r3, W_rel3, b_rel3, C1, C2, C3, C4, bc)

    g = s.reshape(N // 32, 32)
    return _t4(g, W_m1, b_m1, W_m2, b_m2, W_m3, b_m3)
